# baseline (speedup 1.0000x reference)
import dataclasses
import functools

import numpy as np
import jax
import jax.numpy as jnp
from jax import lax
from jax.experimental import pallas as pl
from jax.experimental.pallas import tpu as pltpu
from jax.experimental.pallas import tpu_sc as plsc

NUM_ENT = 10000
NUM_REL = 200
E = 320000
D = 128
NF = 200
KS = 7
K_W = 8
K_H = 16
OWOH = 100
FLAT = NF * OWOH
FLATP = 20480
B = 1024

NPAD = 10240
NTILE = 16
BLK = 32
BLKD = 128
EPT = 20096
EPD = NTILE * EPT
ROWS_PT = NPAD // NTILE

_F32 = jnp.float32
_HI = lax.Precision.HIGHEST


def _sc_compiler_params():
    cp = pltpu.CompilerParams()
    if "needs_layout_passes" in pltpu.CompilerParams.__dataclass_fields__:
        cp = dataclasses.replace(cp, needs_layout_passes=False)
    return cp


def _sc_deg(rows):
    mesh = plsc.VectorSubcoreMesh(core_axis_name="c", subcore_axis_name="s")
    nblk = EPT // BLKD

    @functools.partial(
        pl.kernel,
        out_type=jax.ShapeDtypeStruct((2 * NPAD,), _F32),
        mesh=mesh,
        compiler_params=_sc_compiler_params(),
        scratch_types=[
            pltpu.VMEM((BLKD,), jnp.int32),
            pltpu.VMEM((BLKD,), _F32),
            pltpu.VMEM((ROWS_PT,), _F32),
            pltpu.VMEM_SHARED((NPAD,), _F32),
        ],
    )
    def k(rows_hbm, deg_hbm, row_b, ones_b, z_b, shared_deg):
        c = lax.axis_index("c")
        s = lax.axis_index("s")
        base = c * EPD + s * EPT
        z16 = jnp.zeros((16,), _F32)

        @pl.loop(0, BLKD, step=16)
        def _(i):
            ones_b[pl.ds(i, 16)] = jnp.full((16,), 1.0, _F32)

        @pl.loop(0, ROWS_PT, step=16)
        def _(i):
            z_b[pl.ds(i, 16)] = z16

        pltpu.sync_copy(z_b, shared_deg.at[pl.ds(s * ROWS_PT, ROWS_PT)])
        plsc.subcore_barrier()

        @pl.loop(0, nblk)
        def _(b):
            pltpu.sync_copy(rows_hbm.at[pl.ds(base + b * BLKD, BLKD)], row_b)
            pltpu.sync_copy(ones_b, shared_deg.at[row_b], add=True)

        plsc.subcore_barrier()
        pltpu.sync_copy(shared_deg.at[pl.ds(s * ROWS_PT, ROWS_PT)],
                        deg_hbm.at[pl.ds(c * NPAD + s * ROWS_PT, ROWS_PT)])

    return k(rows)


def _sc_msg(rows, cols, ets, ee, xs, rels):
    mesh = plsc.VectorSubcoreMesh(core_axis_name="c", subcore_axis_name="s")
    nblk = EPT // BLK

    nsets = 2

    @functools.partial(
        pl.kernel,
        out_type=jax.ShapeDtypeStruct((2 * NPAD, D), _F32),
        mesh=mesh,
        compiler_params=_sc_compiler_params(),
        scratch_types=(
            [pltpu.VMEM((BLK,), jnp.int32) for _ in range(3 * nsets)]
            + [pltpu.VMEM((BLK,), jnp.int32) for _ in range(nsets)]
            + [pltpu.VMEM((BLK, D), _F32) for _ in range(4 * nsets)]
            + [pltpu.VMEM_SHARED((NPAD, D), _F32)]
            + [pltpu.SemaphoreType.DMA for _ in range(3 * nsets)]
        ),
    )
    def k(rows_hbm, cols_hbm, ets_hbm, ee_hbm, xs_hbm, rels_hbm, agg_hbm,
          row0, col0, et0, row1, col1, et1, colsc0, colsc1,
          xg0, rg0, eg0, xm0, xg1, rg1, eg1, xm1,
          shared_agg,
          sem_i0, sem_i1, sem_g0, sem_g1, sem_s0, sem_s1):
        c = lax.axis_index("c")
        s = lax.axis_index("s")
        base = c * EPD + s * EPT
        sets = ((row0, col0, et0, colsc0, xg0, rg0, eg0, xm0, sem_i0, sem_g0, sem_s0),
                (row1, col1, et1, colsc1, xg1, rg1, eg1, xm1, sem_i1, sem_g1, sem_s1))
        z16 = jnp.zeros((16,), _F32)

        @pl.loop(0, BLK)
        def _(r):
            for dch in range(D // 16):
                xm0[r, pl.ds(dch * 16, 16)] = z16

        for kk in range(ROWS_PT // BLK):
            pltpu.sync_copy(xm0, shared_agg.at[pl.ds(s * ROWS_PT + kk * BLK, BLK)])
        plsc.subcore_barrier()

        def idx_start(q, b):
            row_b, col_b, et_b, _, _, _, _, _, sem_i, _, _ = sets[q]
            off = base + jnp.minimum(b, nblk - 1) * BLK
            pltpu.async_copy(rows_hbm.at[pl.ds(off, BLK)], row_b, sem_i)
            pltpu.async_copy(cols_hbm.at[pl.ds(off, BLK)], col_b, sem_i)
            pltpu.async_copy(ets_hbm.at[pl.ds(off, BLK)], et_b, sem_i)

        def idx_wait(q, b):
            row_b, col_b, et_b, _, _, _, _, _, sem_i, _, _ = sets[q]
            off = base + jnp.minimum(b, nblk - 1) * BLK
            pltpu.make_async_copy(rows_hbm.at[pl.ds(off, BLK)], row_b, sem_i).wait()
            pltpu.make_async_copy(cols_hbm.at[pl.ds(off, BLK)], col_b, sem_i).wait()
            pltpu.make_async_copy(ets_hbm.at[pl.ds(off, BLK)], et_b, sem_i).wait()

        def gather_start(q, b):
            row_b, _, et_b, _, xg, rg, eg, _, _, sem_g, _ = sets[q]
            pltpu.async_copy(xs_hbm.at[row_b], xg, sem_g)
            pltpu.async_copy(rels_hbm.at[et_b], rg, sem_g)
            eoff = c * E + jnp.minimum(s * EPT + b * BLK, E - BLK)
            pltpu.async_copy(ee_hbm.at[pl.ds(eoff, BLK)], eg, sem_g)

        def gather_wait(q, b):
            row_b, _, et_b, _, xg, rg, eg, _, _, sem_g, _ = sets[q]
            pltpu.make_async_copy(xs_hbm.at[row_b], xg, sem_g).wait()
            pltpu.make_async_copy(rels_hbm.at[et_b], rg, sem_g).wait()
            eoff = c * E + jnp.minimum(s * EPT + b * BLK, E - BLK)
            pltpu.make_async_copy(ee_hbm.at[pl.ds(eoff, BLK)], eg, sem_g).wait()

        def colsc_save(q):
            _, col_b, _, colsc, _, _, _, _, _, _, _ = sets[q]
            @pl.loop(0, BLK, step=16)
            def _(i):
                colsc[pl.ds(i, 16)] = col_b[pl.ds(i, 16)]

        def scatter_start(q):
            _, _, _, colsc, _, _, _, xm, _, _, sem_s = sets[q]
            pltpu.async_copy(xm, shared_agg.at[colsc], sem_s, add=True)

        def scatter_wait(q):
            _, _, _, colsc, _, _, _, xm, _, _, sem_s = sets[q]
            pltpu.make_async_copy(xm, shared_agg.at[colsc], sem_s).wait()

        def compute(q):
            _, _, _, _, xg, rg, eg, xm, _, _, _ = sets[q]
            @pl.loop(0, BLK)
            def _(r):
                for dch in range(D // 16):
                    sl = pl.ds(dch * 16, 16)
                    xm[r, sl] = xg[r, sl] * rg[r, sl] * eg[r, sl]

        for q in range(2):
            idx_start(q, q)
            idx_wait(q, q)
            gather_start(q, q)

        @pl.loop(0, nblk // 2)
        def _(bb):
            b0 = 2 * bb
            for q in range(2):
                b = b0 + q

                @pl.when(bb > 0)
                def _():
                    scatter_wait(q)
                colsc_save(q)
                gather_wait(q, b)
                idx_start(q, b + 2)
                compute(q)
                scatter_start(q)
                idx_wait(q, b + 2)

                @pl.when(b + 2 < nblk)
                def _():
                    gather_start(q, b + 2)

        scatter_wait(0)
        scatter_wait(1)

        plsc.subcore_barrier()
        for kk in range(ROWS_PT // BLK):
            r0 = s * ROWS_PT + kk * BLK
            pltpu.sync_copy(shared_agg.at[pl.ds(r0, BLK)],
                            agg_hbm.at[pl.ds(c * NPAD + r0, BLK)])

    return k(rows, cols, ets, ee, xs, rels)


def _sc_gather(all_ent, all_rel, src, rel):
    mesh = plsc.VectorSubcoreMesh(core_axis_name="c", subcore_axis_name="s")
    rows_pw = B // 32

    @functools.partial(
        pl.kernel,
        out_type=(jax.ShapeDtypeStruct((B, D), _F32),
                  jax.ShapeDtypeStruct((B, D), _F32)),
        mesh=mesh,
        scratch_types=[
            pltpu.VMEM((rows_pw,), jnp.int32),
            pltpu.VMEM((rows_pw, D), _F32),
            pltpu.SemaphoreType.DMA,
        ],
    )
    def k(ae_hbm, ar_hbm, src_hbm, rel_hbm, se_hbm, re_hbm, idx_v, rows_v, sem):
        c = lax.axis_index("c")
        s = lax.axis_index("s")
        wid = s * 2 + c
        b0 = wid * rows_pw
        pltpu.sync_copy(src_hbm.at[pl.ds(b0, rows_pw)], idx_v)
        pltpu.async_copy(ae_hbm.at[idx_v], rows_v, sem).wait()
        pltpu.sync_copy(rows_v, se_hbm.at[pl.ds(b0, rows_pw)])
        pltpu.sync_copy(rel_hbm.at[pl.ds(b0, rows_pw)], idx_v)
        pltpu.async_copy(ar_hbm.at[idx_v], rows_v, sem).wait()
        pltpu.sync_copy(rows_v, re_hbm.at[pl.ds(b0, rows_pw)])

    return k(all_ent, all_rel, src, rel)


def _tc_dinvscale(deg2, xp):
    nb = 2 * NPAD // 1024

    def body(deg_ref, x_ref, xs_ref, dv_ref):
        dg = deg_ref[...]
        dv = jnp.where(dg > 0.5, lax.rsqrt(jnp.maximum(dg, 1e-12)), 0.0)
        dv_ref[...] = dv
        xs_ref[...] = x_ref[...] * dv

    return pl.pallas_call(
        body,
        grid=(nb,),
        in_specs=[pl.BlockSpec((1024, 1), lambda i: (i, 0)),
                  pl.BlockSpec((1024, D), lambda i: (i % (NPAD // 1024), 0))],
        out_specs=[pl.BlockSpec((1024, D), lambda i: (i, 0)),
                   pl.BlockSpec((1024, 1), lambda i: (i, 0))],
        out_shape=[jax.ShapeDtypeStruct((2 * NPAD, D), _F32),
                   jax.ShapeDtypeStruct((2 * NPAD, 1), _F32)],
    )(deg2, xp)


def _tc_encoder(agg_in, agg_out, dv0, dv1, xp, coeff, wi, wo, wl):
    nb = NPAD // 1024

    def body(ai_ref, ao_ref, d0_ref, d1_ref, x_ref, cf_ref, wi_ref, wo_ref,
             wl_ref, pre_ref, st_ref):
        i = pl.program_id(0)
        xc = x_ref[...] * cf_ref[...]
        pre = (jnp.dot(ai_ref[...] * d0_ref[...], wi_ref[...],
                       preferred_element_type=_F32, precision=_HI)
               + jnp.dot(ao_ref[...] * d1_ref[...], wo_ref[...],
                         preferred_element_type=_F32, precision=_HI)
               + jnp.dot(xc, wl_ref[...], preferred_element_type=_F32,
                         precision=_HI)) * (1.0 / 3.0)
        pre_ref[...] = pre

        @pl.when(i == 0)
        def _():
            st_ref[...] = jnp.zeros_like(st_ref)

        st_ref[0:1, :] += jnp.sum(pre, axis=0, keepdims=True)
        st_ref[1:2, :] += jnp.sum(pre * pre, axis=0, keepdims=True)

    blk = pl.BlockSpec((1024, D), lambda i: (i, 0))
    dblk = pl.BlockSpec((1024, 1), lambda i: (i, 0))
    wblk = pl.BlockSpec((D, D), lambda i: (0, 0))
    return pl.pallas_call(
        body,
        grid=(nb,),
        in_specs=[blk, blk, dblk, dblk, blk, pl.BlockSpec((1, D), lambda i: (0, 0)),
                  wblk, wblk, wblk],
        out_specs=[blk, pl.BlockSpec((8, D), lambda i: (0, 0))],
        out_shape=[jax.ShapeDtypeStruct((NPAD, D), _F32),
                   jax.ShapeDtypeStruct((8, D), _F32)],
    )(agg_in, agg_out, dv0, dv1, xp, coeff, wi, wo, wl)


def _tc_entnorm(pre, bnpack):
    nb = NPAD // 1024

    def body(pre_ref, bn_ref, ae_ref):
        mean = bn_ref[0:1, :]
        gs = bn_ref[1:2, :]
        bb = bn_ref[2:3, :]
        ae_ref[...] = jnp.tanh((pre_ref[...] - mean) * gs + bb)

    blk = pl.BlockSpec((1024, D), lambda i: (i, 0))
    return pl.pallas_call(
        body,
        grid=(nb,),
        in_specs=[blk, pl.BlockSpec((8, D), lambda i: (0, 0))],
        out_specs=blk,
        out_shape=jax.ShapeDtypeStruct((NPAD, D), _F32),
    )(pre, bnpack)


def _tc_relmm(rels_pad, wr):
    def body(r_ref, w_ref, o_ref):
        o_ref[...] = jnp.dot(r_ref[...], w_ref[...], preferred_element_type=_F32,
                             precision=_HI)

    return pl.pallas_call(
        body,
        out_shape=jax.ShapeDtypeStruct((512, D), _F32),
    )(rels_pad, wr)


def _tc_convmm(cat, wc):
    nb = FLATP // 128

    def body(cat_ref, wc_ref, raw_ref, st_ref, cst_ref):
        j = pl.program_id(0)
        catv = cat_ref[...]
        raw = jnp.dot(catv, wc_ref[...], preferred_element_type=_F32,
                      precision=_HI)
        raw_ref[...] = raw
        st_ref[...] = jnp.concatenate(
            [jnp.sum(raw, axis=0, keepdims=True),
             jnp.sum(raw * raw, axis=0, keepdims=True),
             jnp.zeros((6, 128), _F32)], axis=0)

        @pl.when(j == 0)
        def _():
            cst_ref[...] = jnp.concatenate(
                [jnp.sum(catv, axis=0, keepdims=True),
                 jnp.sum(catv * catv, axis=0, keepdims=True),
                 jnp.zeros((6, 2 * D), _F32)], axis=0)

    return pl.pallas_call(
        body,
        grid=(nb,),
        in_specs=[pl.BlockSpec((B, 2 * D), lambda j: (0, 0)),
                  pl.BlockSpec((2 * D, 128), lambda j: (0, j))],
        out_specs=[pl.BlockSpec((B, 128), lambda j: (0, j)),
                   pl.BlockSpec((8, 128), lambda j: (0, j)),
                   pl.BlockSpec((8, 2 * D), lambda j: (0, 0))],
        out_shape=[jax.ShapeDtypeStruct((B, FLATP), _F32),
                   jax.ShapeDtypeStruct((8, FLATP), _F32),
                   jax.ShapeDtypeStruct((8, 2 * D), _F32)],
    )(cat, wc)


def _tc_fc(raw, ab, fcw2, fcb):
    nb = FLATP // 128

    def body(raw_ref, ab_ref, w_ref, fcb_ref, h_ref, st_ref):
        j = pl.program_id(0)
        h1 = jnp.maximum(raw_ref[...] * ab_ref[0:1, :] + ab_ref[1:2, :], 0.0)

        @pl.when(j == 0)
        def _():
            h_ref[...] = jnp.broadcast_to(fcb_ref[...], (B, D))

        h_ref[...] += jnp.dot(h1, w_ref[...], preferred_element_type=_F32,
                              precision=_HI)

        @pl.when(j == nb - 1)
        def _():
            h = h_ref[...]
            st_ref[...] = jnp.concatenate(
                [jnp.sum(h, axis=0, keepdims=True),
                 jnp.sum(h * h, axis=0, keepdims=True),
                 jnp.zeros((6, D), _F32)], axis=0)

    return pl.pallas_call(
        body,
        grid=(nb,),
        in_specs=[pl.BlockSpec((B, 128), lambda j: (0, j)),
                  pl.BlockSpec((8, 128), lambda j: (0, j)),
                  pl.BlockSpec((128, D), lambda j: (j, 0)),
                  pl.BlockSpec((1, D), lambda j: (0, 0))],
        out_specs=[pl.BlockSpec((B, D), lambda j: (0, 0)),
                   pl.BlockSpec((8, D), lambda j: (0, 0))],
        out_shape=[jax.ShapeDtypeStruct((B, D), _F32),
                   jax.ShapeDtypeStruct((8, D), _F32)],
    )(raw, ab, fcw2, fcb)


def _tc_score(h, bn2pack, all_ent, bias):
    nb = NPAD // 1024

    def body(h_ref, bn_ref, ae_ref, bias_ref, o_ref):
        h2 = jnp.maximum((h_ref[...] - bn_ref[0:1, :]) * bn_ref[1:2, :]
                         + bn_ref[2:3, :], 0.0)
        logits = lax.dot_general(h2, ae_ref[...], (((1,), (1,)), ((), ())),
                                 preferred_element_type=_F32, precision=_HI)
        o_ref[...] = jax.nn.sigmoid(logits + bias_ref[...])

    return pl.pallas_call(
        body,
        grid=(nb,),
        in_specs=[pl.BlockSpec((B, D), lambda j: (0, 0)),
                  pl.BlockSpec((8, D), lambda j: (0, 0)),
                  pl.BlockSpec((1024, D), lambda j: (j, 0)),
                  pl.BlockSpec((1, 1024), lambda j: (0, j))],
        out_specs=pl.BlockSpec((B, 1024), lambda j: (0, j)),
        out_shape=jax.ShapeDtypeStruct((B, NPAD), _F32),
    )(h, bn2pack, all_ent, bias)


def _conv_qidx():
    qidx = np.full((2 * D, OWOH), KS * KS, np.int32)
    for rowcat in range(2 * D):
        c, dd = rowcat // D, rowcat % D
        p = 2 * dd + c
        pi, pj = p // K_H, p % K_H
        for ij in range(OWOH):
            i, j = ij // 10, ij % 10
            ki, kj = pi - i, pj - j
            if 0 <= ki < KS and 0 <= kj < KS:
                qidx[rowcat, ij] = ki * KS + kj
    return qidx.reshape(-1)

_QIDX = _conv_qidx()


def kernel(src, rel, entity, edge_index, edge_norm, edge_type, edge_ids,
           entity_embedding, relation_embedding, edge_embeddings,
           in_weight, out_weight, loop_weight, rels_weight, loop_rel, loop_edge,
           ent_bn_g, ent_bn_b, bn0_g, bn0_b, bn1_g, bn1_b, bn2_g, bn2_b,
           conv_w, fc_w, fc_b, ent_bias):
    f32 = _F32
    half = E

    ei = edge_index.astype(jnp.int32)
    et = edge_type.astype(jnp.int32)

    def _pad_edges(v, fill):
        return jnp.pad(v, (0, EPD - E), constant_values=fill)

    rows_in = _pad_edges(ei[0, :half], NPAD - 1)
    rows_out = _pad_edges(ei[0, half:], NPAD - 1)
    rows = jnp.concatenate([rows_in, rows_out])
    rows_off = jnp.concatenate([rows_in, rows_out + NPAD])
    cols = jnp.concatenate([_pad_edges(ei[1, :half], NPAD - 1),
                            _pad_edges(ei[1, half:], NPAD - 1)])
    ets = jnp.concatenate([_pad_edges(et[:half], 0), _pad_edges(et[half:], 0)])

    xp = jnp.pad(entity_embedding.astype(f32), ((0, NPAD - NUM_ENT), (0, 0)))
    rels = jnp.concatenate([relation_embedding, loop_rel], axis=0).astype(f32)
    rels_pad = jnp.pad(rels, ((0, 512 - rels.shape[0]), (0, 0)))

    deg = _sc_deg(rows)
    xs, dvc = _tc_dinvscale(deg.reshape(2 * NPAD, 1), xp)

    agg = _sc_msg(rows_off, cols, ets, edge_embeddings.astype(f32), xs, rels_pad)
    agg_in, agg_out = agg[:NPAD], agg[NPAD:]

    coeff = (loop_rel * loop_edge).astype(f32)
    pre, st = _tc_encoder(agg_in, agg_out, dvc[:NPAD], dvc[NPAD:], xp, coeff,
                          in_weight.astype(f32), out_weight.astype(f32),
                          loop_weight.astype(f32))
    m = st[0] / NUM_ENT
    v = st[1] / NUM_ENT - m * m
    gs = ent_bn_g / jnp.sqrt(v + 1e-5)
    bnpack = jnp.zeros((8, D), f32).at[0].set(m).at[1].set(gs).at[2].set(ent_bn_b)
    all_ent = _tc_entnorm(pre, bnpack)

    all_rel = _tc_relmm(rels_pad, rels_weight.astype(f32))

    src_emb, rel_emb = _sc_gather(all_ent, all_rel, src.astype(jnp.int32),
                                  rel.astype(jnp.int32))
    cat = jnp.concatenate([src_emb, rel_emb], axis=1)

    cwT = conv_w.reshape(NF, KS * KS).T.astype(f32)
    cwT_ext = jnp.concatenate([cwT, jnp.zeros((1, NF), f32)], axis=0)
    wc = jnp.take(cwT_ext, jnp.asarray(_QIDX), axis=0).reshape(2 * D, FLAT)
    wc = jnp.pad(wc, ((0, 0), (0, FLATP - FLAT)))
    wsum = jnp.sum(wc, axis=0)

    raw, rst, cst = _tc_convmm(cat, wc)

    n0 = B * 2 * D
    s0 = jnp.sum(cst[0])
    s0sq = jnp.sum(cst[1])
    m0 = s0 / n0
    v0 = s0sq / n0 - m0 * m0
    a0 = bn0_g[0] / jnp.sqrt(v0 + 1e-5)
    c0 = bn0_b[0] - m0 * a0

    csum, csumsq = rst[0], rst[1]
    co_sum = a0 * csum + B * c0 * wsum
    co_sumsq = (a0 * a0 * csumsq + 2 * a0 * c0 * wsum * csum
                + B * (c0 * wsum) ** 2)
    g_sum = co_sum[:FLAT].reshape(OWOH, NF).sum(0)
    g_sumsq = co_sumsq[:FLAT].reshape(OWOH, NF).sum(0)
    n1 = B * OWOH
    m1 = g_sum / n1
    v1 = g_sumsq / n1 - m1 * m1
    alpha_f = bn1_g / jnp.sqrt(v1 + 1e-5)
    beta_f = bn1_b - m1 * alpha_f
    alpha = jnp.pad(jnp.tile(alpha_f * a0, OWOH), (0, FLATP - FLAT))
    beta = jnp.pad(jnp.tile(alpha_f, OWOH) * c0 * wsum[:FLAT]
                   + jnp.tile(beta_f, OWOH), (0, FLATP - FLAT))
    ab = jnp.zeros((8, FLATP), f32).at[0].set(alpha).at[1].set(beta)

    fc_w2 = fc_w.reshape(NF, OWOH, D).transpose(1, 0, 2).reshape(FLAT, D)
    fc_w2 = jnp.pad(fc_w2.astype(f32), ((0, FLATP - FLAT), (0, 0)))

    h, hst = _tc_fc(raw, ab, fc_w2, fc_b.reshape(1, D).astype(f32))

    m2 = hst[0] / B
    v2 = hst[1] / B - m2 * m2
    g2s = bn2_g / jnp.sqrt(v2 + 1e-5)
    bn2pack = jnp.zeros((8, D), f32).at[0].set(m2).at[1].set(g2s).at[2].set(bn2_b)

    bias = jnp.pad(ent_bias.reshape(1, NUM_ENT).astype(f32),
                   ((0, 0), (0, NPAD - NUM_ENT)))
    score = _tc_score(h, bn2pack, all_ent, bias)
    return score[:, :NUM_ENT]


# speedup vs baseline: 1.0746x; 1.0746x over previous
import dataclasses
import functools

import numpy as np
import jax
import jax.numpy as jnp
from jax import lax
from jax.experimental import pallas as pl
from jax.experimental.pallas import tpu as pltpu
from jax.experimental.pallas import tpu_sc as plsc

NUM_ENT = 10000
NUM_REL = 200
E = 320000
D = 128
NF = 200
KS = 7
K_W = 8
K_H = 16
OWOH = 100
FLAT = NF * OWOH
FLATP = 20480
B = 1024

NPAD = 10240
NTILE = 16
BLK = 32
BLKD = 128
EPT = 20096
EPD = NTILE * EPT
ROWS_PT = NPAD // NTILE

_F32 = jnp.float32
_HI = lax.Precision.DEFAULT


def _sc_compiler_params():
    cp = pltpu.CompilerParams()
    if "needs_layout_passes" in pltpu.CompilerParams.__dataclass_fields__:
        cp = dataclasses.replace(cp, needs_layout_passes=False)
    return cp


def _sc_deg(rows):
    mesh = plsc.VectorSubcoreMesh(core_axis_name="c", subcore_axis_name="s")
    nblk = EPT // BLKD

    @functools.partial(
        pl.kernel,
        out_type=jax.ShapeDtypeStruct((2 * NPAD,), _F32),
        mesh=mesh,
        compiler_params=_sc_compiler_params(),
        scratch_types=[
            pltpu.VMEM((BLKD,), jnp.int32),
            pltpu.VMEM((BLKD,), _F32),
            pltpu.VMEM((ROWS_PT,), _F32),
            pltpu.VMEM_SHARED((NPAD,), _F32),
        ],
    )
    def k(rows_hbm, deg_hbm, row_b, ones_b, z_b, shared_deg):
        c = lax.axis_index("c")
        s = lax.axis_index("s")
        base = c * EPD + s * EPT
        z16 = jnp.zeros((16,), _F32)

        @pl.loop(0, BLKD, step=16)
        def _(i):
            ones_b[pl.ds(i, 16)] = jnp.full((16,), 1.0, _F32)

        @pl.loop(0, ROWS_PT, step=16)
        def _(i):
            z_b[pl.ds(i, 16)] = z16

        pltpu.sync_copy(z_b, shared_deg.at[pl.ds(s * ROWS_PT, ROWS_PT)])
        plsc.subcore_barrier()

        @pl.loop(0, nblk)
        def _(b):
            pltpu.sync_copy(rows_hbm.at[pl.ds(base + b * BLKD, BLKD)], row_b)
            pltpu.sync_copy(ones_b, shared_deg.at[row_b], add=True)

        plsc.subcore_barrier()
        pltpu.sync_copy(shared_deg.at[pl.ds(s * ROWS_PT, ROWS_PT)],
                        deg_hbm.at[pl.ds(c * NPAD + s * ROWS_PT, ROWS_PT)])

    return k(rows)


def _sc_msg(rows, cols, ets, ee, xs, rels):
    mesh = plsc.VectorSubcoreMesh(core_axis_name="c", subcore_axis_name="s")
    nblk = EPT // BLK

    nsets = 2

    @functools.partial(
        pl.kernel,
        out_type=jax.ShapeDtypeStruct((2 * NPAD, D), _F32),
        mesh=mesh,
        compiler_params=_sc_compiler_params(),
        scratch_types=(
            [pltpu.VMEM((BLK,), jnp.int32) for _ in range(3 * nsets)]
            + [pltpu.VMEM((BLK,), jnp.int32) for _ in range(nsets)]
            + [pltpu.VMEM((BLK, D), _F32) for _ in range(4 * nsets)]
            + [pltpu.VMEM_SHARED((NPAD, D), _F32)]
            + [pltpu.SemaphoreType.DMA for _ in range(3 * nsets)]
        ),
    )
    def k(rows_hbm, cols_hbm, ets_hbm, ee_hbm, xs_hbm, rels_hbm, agg_hbm,
          row0, col0, et0, row1, col1, et1, colsc0, colsc1,
          xg0, rg0, eg0, xm0, xg1, rg1, eg1, xm1,
          shared_agg,
          sem_i0, sem_i1, sem_g0, sem_g1, sem_s0, sem_s1):
        c = lax.axis_index("c")
        s = lax.axis_index("s")
        base = c * EPD + s * EPT
        sets = ((row0, col0, et0, colsc0, xg0, rg0, eg0, xm0, sem_i0, sem_g0, sem_s0),
                (row1, col1, et1, colsc1, xg1, rg1, eg1, xm1, sem_i1, sem_g1, sem_s1))
        z16 = jnp.zeros((16,), _F32)

        @pl.loop(0, BLK)
        def _(r):
            for dch in range(D // 16):
                xm0[r, pl.ds(dch * 16, 16)] = z16

        for kk in range(ROWS_PT // BLK):
            pltpu.sync_copy(xm0, shared_agg.at[pl.ds(s * ROWS_PT + kk * BLK, BLK)])
        plsc.subcore_barrier()

        def idx_start(q, b):
            row_b, col_b, et_b, _, _, _, _, _, sem_i, _, _ = sets[q]
            off = base + jnp.minimum(b, nblk - 1) * BLK
            pltpu.async_copy(rows_hbm.at[pl.ds(off, BLK)], row_b, sem_i)
            pltpu.async_copy(cols_hbm.at[pl.ds(off, BLK)], col_b, sem_i)
            pltpu.async_copy(ets_hbm.at[pl.ds(off, BLK)], et_b, sem_i)

        def idx_wait(q, b):
            row_b, col_b, et_b, _, _, _, _, _, sem_i, _, _ = sets[q]
            off = base + jnp.minimum(b, nblk - 1) * BLK
            pltpu.make_async_copy(rows_hbm.at[pl.ds(off, BLK)], row_b, sem_i).wait()
            pltpu.make_async_copy(cols_hbm.at[pl.ds(off, BLK)], col_b, sem_i).wait()
            pltpu.make_async_copy(ets_hbm.at[pl.ds(off, BLK)], et_b, sem_i).wait()

        def gather_start(q, b):
            row_b, _, et_b, _, xg, rg, eg, _, _, sem_g, _ = sets[q]
            pltpu.async_copy(xs_hbm.at[row_b], xg, sem_g)
            pltpu.async_copy(rels_hbm.at[et_b], rg, sem_g)
            eoff = c * E + jnp.minimum(s * EPT + b * BLK, E - BLK)
            pltpu.async_copy(ee_hbm.at[pl.ds(eoff, BLK)], eg, sem_g)

        def gather_wait(q, b):
            row_b, _, et_b, _, xg, rg, eg, _, _, sem_g, _ = sets[q]
            pltpu.make_async_copy(xs_hbm.at[row_b], xg, sem_g).wait()
            pltpu.make_async_copy(rels_hbm.at[et_b], rg, sem_g).wait()
            eoff = c * E + jnp.minimum(s * EPT + b * BLK, E - BLK)
            pltpu.make_async_copy(ee_hbm.at[pl.ds(eoff, BLK)], eg, sem_g).wait()

        def colsc_save(q):
            _, col_b, _, colsc, _, _, _, _, _, _, _ = sets[q]
            @pl.loop(0, BLK, step=16)
            def _(i):
                colsc[pl.ds(i, 16)] = col_b[pl.ds(i, 16)]

        def scatter_start(q):
            _, _, _, colsc, _, _, _, xm, _, _, sem_s = sets[q]
            pltpu.async_copy(xm, shared_agg.at[colsc], sem_s, add=True)

        def scatter_wait(q):
            _, _, _, colsc, _, _, _, xm, _, _, sem_s = sets[q]
            pltpu.make_async_copy(xm, shared_agg.at[colsc], sem_s).wait()

        def compute(q):
            _, _, _, _, xg, rg, eg, xm, _, _, _ = sets[q]
            @pl.loop(0, BLK)
            def _(r):
                for dch in range(D // 16):
                    sl = pl.ds(dch * 16, 16)
                    xm[r, sl] = xg[r, sl] * rg[r, sl] * eg[r, sl]

        for q in range(2):
            idx_start(q, q)
            idx_wait(q, q)
            gather_start(q, q)

        @pl.loop(0, nblk // 2)
        def _(bb):
            b0 = 2 * bb
            for q in range(2):
                b = b0 + q

                @pl.when(bb > 0)
                def _():
                    scatter_wait(q)
                colsc_save(q)
                gather_wait(q, b)
                idx_start(q, b + 2)
                compute(q)
                scatter_start(q)
                idx_wait(q, b + 2)

                @pl.when(b + 2 < nblk)
                def _():
                    gather_start(q, b + 2)

        scatter_wait(0)
        scatter_wait(1)

        plsc.subcore_barrier()
        for kk in range(ROWS_PT // BLK):
            r0 = s * ROWS_PT + kk * BLK
            pltpu.sync_copy(shared_agg.at[pl.ds(r0, BLK)],
                            agg_hbm.at[pl.ds(c * NPAD + r0, BLK)])

    return k(rows, cols, ets, ee, xs, rels)


def _sc_gather(all_ent, all_rel, src, rel):
    mesh = plsc.VectorSubcoreMesh(core_axis_name="c", subcore_axis_name="s")
    rows_pw = B // 32

    @functools.partial(
        pl.kernel,
        out_type=(jax.ShapeDtypeStruct((B, D), _F32),
                  jax.ShapeDtypeStruct((B, D), _F32)),
        mesh=mesh,
        scratch_types=[
            pltpu.VMEM((rows_pw,), jnp.int32),
            pltpu.VMEM((rows_pw, D), _F32),
            pltpu.SemaphoreType.DMA,
        ],
    )
    def k(ae_hbm, ar_hbm, src_hbm, rel_hbm, se_hbm, re_hbm, idx_v, rows_v, sem):
        c = lax.axis_index("c")
        s = lax.axis_index("s")
        wid = s * 2 + c
        b0 = wid * rows_pw
        pltpu.sync_copy(src_hbm.at[pl.ds(b0, rows_pw)], idx_v)
        pltpu.async_copy(ae_hbm.at[idx_v], rows_v, sem).wait()
        pltpu.sync_copy(rows_v, se_hbm.at[pl.ds(b0, rows_pw)])
        pltpu.sync_copy(rel_hbm.at[pl.ds(b0, rows_pw)], idx_v)
        pltpu.async_copy(ar_hbm.at[idx_v], rows_v, sem).wait()
        pltpu.sync_copy(rows_v, re_hbm.at[pl.ds(b0, rows_pw)])

    return k(all_ent, all_rel, src, rel)


def _tc_dinvscale(deg2, xp):
    nb = 2 * NPAD // 1024

    def body(deg_ref, x_ref, xs_ref, dv_ref):
        dg = deg_ref[...]
        dv = jnp.where(dg > 0.5, lax.rsqrt(jnp.maximum(dg, 1e-12)), 0.0)
        dv_ref[...] = dv
        xs_ref[...] = x_ref[...] * dv

    return pl.pallas_call(
        body,
        grid=(nb,),
        in_specs=[pl.BlockSpec((1024, 1), lambda i: (i, 0)),
                  pl.BlockSpec((1024, D), lambda i: (i % (NPAD // 1024), 0))],
        out_specs=[pl.BlockSpec((1024, D), lambda i: (i, 0)),
                   pl.BlockSpec((1024, 1), lambda i: (i, 0))],
        out_shape=[jax.ShapeDtypeStruct((2 * NPAD, D), _F32),
                   jax.ShapeDtypeStruct((2 * NPAD, 1), _F32)],
    )(deg2, xp)


def _tc_encoder(agg_in, agg_out, dv0, dv1, xp, coeff, wi, wo, wl):
    nb = NPAD // 1024

    def body(ai_ref, ao_ref, d0_ref, d1_ref, x_ref, cf_ref, wi_ref, wo_ref,
             wl_ref, pre_ref, st_ref):
        i = pl.program_id(0)
        xc = x_ref[...] * cf_ref[...]
        pre = (jnp.dot(ai_ref[...] * d0_ref[...], wi_ref[...],
                       preferred_element_type=_F32, precision=_HI)
               + jnp.dot(ao_ref[...] * d1_ref[...], wo_ref[...],
                         preferred_element_type=_F32, precision=_HI)
               + jnp.dot(xc, wl_ref[...], preferred_element_type=_F32,
                         precision=_HI)) * (1.0 / 3.0)
        pre_ref[...] = pre

        @pl.when(i == 0)
        def _():
            st_ref[...] = jnp.zeros_like(st_ref)

        st_ref[0:1, :] += jnp.sum(pre, axis=0, keepdims=True)
        st_ref[1:2, :] += jnp.sum(pre * pre, axis=0, keepdims=True)

    blk = pl.BlockSpec((1024, D), lambda i: (i, 0))
    dblk = pl.BlockSpec((1024, 1), lambda i: (i, 0))
    wblk = pl.BlockSpec((D, D), lambda i: (0, 0))
    return pl.pallas_call(
        body,
        grid=(nb,),
        in_specs=[blk, blk, dblk, dblk, blk, pl.BlockSpec((1, D), lambda i: (0, 0)),
                  wblk, wblk, wblk],
        out_specs=[blk, pl.BlockSpec((8, D), lambda i: (0, 0))],
        out_shape=[jax.ShapeDtypeStruct((NPAD, D), _F32),
                   jax.ShapeDtypeStruct((8, D), _F32)],
    )(agg_in, agg_out, dv0, dv1, xp, coeff, wi, wo, wl)


def _tc_entnorm(pre, bnpack):
    nb = NPAD // 1024

    def body(pre_ref, bn_ref, ae_ref):
        mean = bn_ref[0:1, :]
        gs = bn_ref[1:2, :]
        bb = bn_ref[2:3, :]
        ae_ref[...] = jnp.tanh((pre_ref[...] - mean) * gs + bb)

    blk = pl.BlockSpec((1024, D), lambda i: (i, 0))
    return pl.pallas_call(
        body,
        grid=(nb,),
        in_specs=[blk, pl.BlockSpec((8, D), lambda i: (0, 0))],
        out_specs=blk,
        out_shape=jax.ShapeDtypeStruct((NPAD, D), _F32),
    )(pre, bnpack)


def _tc_relmm(rels_pad, wr):
    def body(r_ref, w_ref, o_ref):
        o_ref[...] = jnp.dot(r_ref[...], w_ref[...], preferred_element_type=_F32,
                             precision=_HI)

    return pl.pallas_call(
        body,
        out_shape=jax.ShapeDtypeStruct((512, D), _F32),
    )(rels_pad, wr)


def _tc_convmm(cat, wc):
    nb = FLATP // 128

    def body(cat_ref, wc_ref, raw_ref, st_ref, cst_ref):
        j = pl.program_id(0)
        catv = cat_ref[...]
        raw = jnp.dot(catv, wc_ref[...], preferred_element_type=_F32,
                      precision=_HI)
        raw_ref[...] = raw
        st_ref[...] = jnp.concatenate(
            [jnp.sum(raw, axis=0, keepdims=True),
             jnp.sum(raw * raw, axis=0, keepdims=True),
             jnp.zeros((6, 128), _F32)], axis=0)

        @pl.when(j == 0)
        def _():
            cst_ref[...] = jnp.concatenate(
                [jnp.sum(catv, axis=0, keepdims=True),
                 jnp.sum(catv * catv, axis=0, keepdims=True),
                 jnp.zeros((6, 2 * D), _F32)], axis=0)

    return pl.pallas_call(
        body,
        grid=(nb,),
        in_specs=[pl.BlockSpec((B, 2 * D), lambda j: (0, 0)),
                  pl.BlockSpec((2 * D, 128), lambda j: (0, j))],
        out_specs=[pl.BlockSpec((B, 128), lambda j: (0, j)),
                   pl.BlockSpec((8, 128), lambda j: (0, j)),
                   pl.BlockSpec((8, 2 * D), lambda j: (0, 0))],
        out_shape=[jax.ShapeDtypeStruct((B, FLATP), _F32),
                   jax.ShapeDtypeStruct((8, FLATP), _F32),
                   jax.ShapeDtypeStruct((8, 2 * D), _F32)],
    )(cat, wc)


def _tc_fc(raw, ab, fcw2, fcb):
    nb = FLATP // 128

    def body(raw_ref, ab_ref, w_ref, fcb_ref, h_ref, st_ref):
        j = pl.program_id(0)
        h1 = jnp.maximum(raw_ref[...] * ab_ref[0:1, :] + ab_ref[1:2, :], 0.0)

        @pl.when(j == 0)
        def _():
            h_ref[...] = jnp.broadcast_to(fcb_ref[...], (B, D))

        h_ref[...] += jnp.dot(h1, w_ref[...], preferred_element_type=_F32,
                              precision=_HI)

        @pl.when(j == nb - 1)
        def _():
            h = h_ref[...]
            st_ref[...] = jnp.concatenate(
                [jnp.sum(h, axis=0, keepdims=True),
                 jnp.sum(h * h, axis=0, keepdims=True),
                 jnp.zeros((6, D), _F32)], axis=0)

    return pl.pallas_call(
        body,
        grid=(nb,),
        in_specs=[pl.BlockSpec((B, 128), lambda j: (0, j)),
                  pl.BlockSpec((8, 128), lambda j: (0, j)),
                  pl.BlockSpec((128, D), lambda j: (j, 0)),
                  pl.BlockSpec((1, D), lambda j: (0, 0))],
        out_specs=[pl.BlockSpec((B, D), lambda j: (0, 0)),
                   pl.BlockSpec((8, D), lambda j: (0, 0))],
        out_shape=[jax.ShapeDtypeStruct((B, D), _F32),
                   jax.ShapeDtypeStruct((8, D), _F32)],
    )(raw, ab, fcw2, fcb)


def _tc_score(h, bn2pack, all_ent, bias):
    nb = NPAD // 1024

    def body(h_ref, bn_ref, ae_ref, bias_ref, o_ref):
        h2 = jnp.maximum((h_ref[...] - bn_ref[0:1, :]) * bn_ref[1:2, :]
                         + bn_ref[2:3, :], 0.0)
        logits = lax.dot_general(h2, ae_ref[...], (((1,), (1,)), ((), ())),
                                 preferred_element_type=_F32, precision=_HI)
        o_ref[...] = jax.nn.sigmoid(logits + bias_ref[...])

    return pl.pallas_call(
        body,
        grid=(nb,),
        in_specs=[pl.BlockSpec((B, D), lambda j: (0, 0)),
                  pl.BlockSpec((8, D), lambda j: (0, 0)),
                  pl.BlockSpec((1024, D), lambda j: (j, 0)),
                  pl.BlockSpec((1, 1024), lambda j: (0, j))],
        out_specs=pl.BlockSpec((B, 1024), lambda j: (0, j)),
        out_shape=jax.ShapeDtypeStruct((B, NPAD), _F32),
    )(h, bn2pack, all_ent, bias)


def _conv_qidx():
    qidx = np.full((2 * D, OWOH), KS * KS, np.int32)
    for rowcat in range(2 * D):
        c, dd = rowcat // D, rowcat % D
        p = 2 * dd + c
        pi, pj = p // K_H, p % K_H
        for ij in range(OWOH):
            i, j = ij // 10, ij % 10
            ki, kj = pi - i, pj - j
            if 0 <= ki < KS and 0 <= kj < KS:
                qidx[rowcat, ij] = ki * KS + kj
    return qidx.reshape(-1)

_QIDX = _conv_qidx()


def kernel(src, rel, entity, edge_index, edge_norm, edge_type, edge_ids,
           entity_embedding, relation_embedding, edge_embeddings,
           in_weight, out_weight, loop_weight, rels_weight, loop_rel, loop_edge,
           ent_bn_g, ent_bn_b, bn0_g, bn0_b, bn1_g, bn1_b, bn2_g, bn2_b,
           conv_w, fc_w, fc_b, ent_bias):
    f32 = _F32
    half = E

    ei = edge_index.astype(jnp.int32)
    et = edge_type.astype(jnp.int32)

    def _pad_edges(v, fill):
        return jnp.pad(v, (0, EPD - E), constant_values=fill)

    rows_in = _pad_edges(ei[0, :half], NPAD - 1)
    rows_out = _pad_edges(ei[0, half:], NPAD - 1)
    rows = jnp.concatenate([rows_in, rows_out])
    rows_off = jnp.concatenate([rows_in, rows_out + NPAD])
    cols = jnp.concatenate([_pad_edges(ei[1, :half], NPAD - 1),
                            _pad_edges(ei[1, half:], NPAD - 1)])
    ets = jnp.concatenate([_pad_edges(et[:half], 0), _pad_edges(et[half:], 0)])

    xp = jnp.pad(entity_embedding.astype(f32), ((0, NPAD - NUM_ENT), (0, 0)))
    rels = jnp.concatenate([relation_embedding, loop_rel], axis=0).astype(f32)
    rels_pad = jnp.pad(rels, ((0, 512 - rels.shape[0]), (0, 0)))

    deg = _sc_deg(rows)
    xs, dvc = _tc_dinvscale(deg.reshape(2 * NPAD, 1), xp)

    agg = _sc_msg(rows_off, cols, ets, edge_embeddings.astype(f32), xs, rels_pad)
    agg_in, agg_out = agg[:NPAD], agg[NPAD:]

    coeff = (loop_rel * loop_edge).astype(f32)
    pre, st = _tc_encoder(agg_in, agg_out, dvc[:NPAD], dvc[NPAD:], xp, coeff,
                          in_weight.astype(f32), out_weight.astype(f32),
                          loop_weight.astype(f32))
    m = st[0] / NUM_ENT
    v = st[1] / NUM_ENT - m * m
    gs = ent_bn_g / jnp.sqrt(v + 1e-5)
    bnpack = jnp.zeros((8, D), f32).at[0].set(m).at[1].set(gs).at[2].set(ent_bn_b)
    all_ent = _tc_entnorm(pre, bnpack)

    all_rel = _tc_relmm(rels_pad, rels_weight.astype(f32))

    src_emb, rel_emb = _sc_gather(all_ent, all_rel, src.astype(jnp.int32),
                                  rel.astype(jnp.int32))
    cat = jnp.concatenate([src_emb, rel_emb], axis=1)

    cwT = conv_w.reshape(NF, KS * KS).T.astype(f32)
    cwT_ext = jnp.concatenate([cwT, jnp.zeros((1, NF), f32)], axis=0)
    wc = jnp.take(cwT_ext, jnp.asarray(_QIDX), axis=0).reshape(2 * D, FLAT)
    wc = jnp.pad(wc, ((0, 0), (0, FLATP - FLAT)))
    wsum = jnp.sum(wc, axis=0)

    raw, rst, cst = _tc_convmm(cat, wc)

    n0 = B * 2 * D
    s0 = jnp.sum(cst[0])
    s0sq = jnp.sum(cst[1])
    m0 = s0 / n0
    v0 = s0sq / n0 - m0 * m0
    a0 = bn0_g[0] / jnp.sqrt(v0 + 1e-5)
    c0 = bn0_b[0] - m0 * a0

    csum, csumsq = rst[0], rst[1]
    co_sum = a0 * csum + B * c0 * wsum
    co_sumsq = (a0 * a0 * csumsq + 2 * a0 * c0 * wsum * csum
                + B * (c0 * wsum) ** 2)
    g_sum = co_sum[:FLAT].reshape(OWOH, NF).sum(0)
    g_sumsq = co_sumsq[:FLAT].reshape(OWOH, NF).sum(0)
    n1 = B * OWOH
    m1 = g_sum / n1
    v1 = g_sumsq / n1 - m1 * m1
    alpha_f = bn1_g / jnp.sqrt(v1 + 1e-5)
    beta_f = bn1_b - m1 * alpha_f
    alpha = jnp.pad(jnp.tile(alpha_f * a0, OWOH), (0, FLATP - FLAT))
    beta = jnp.pad(jnp.tile(alpha_f, OWOH) * c0 * wsum[:FLAT]
                   + jnp.tile(beta_f, OWOH), (0, FLATP - FLAT))
    ab = jnp.zeros((8, FLATP), f32).at[0].set(alpha).at[1].set(beta)

    fc_w2 = fc_w.reshape(NF, OWOH, D).transpose(1, 0, 2).reshape(FLAT, D)
    fc_w2 = jnp.pad(fc_w2.astype(f32), ((0, FLATP - FLAT), (0, 0)))

    h, hst = _tc_fc(raw, ab, fc_w2, fc_b.reshape(1, D).astype(f32))

    m2 = hst[0] / B
    v2 = hst[1] / B - m2 * m2
    g2s = bn2_g / jnp.sqrt(v2 + 1e-5)
    bn2pack = jnp.zeros((8, D), f32).at[0].set(m2).at[1].set(g2s).at[2].set(bn2_b)

    bias = jnp.pad(ent_bias.reshape(1, NUM_ENT).astype(f32),
                   ((0, 0), (0, NPAD - NUM_ENT)))
    score = _tc_score(h, bn2pack, all_ent, bias)
    return score[:, :NUM_ENT]


# speedup vs baseline: 1.0767x; 1.0019x over previous
import dataclasses
import functools

import numpy as np
import jax
import jax.numpy as jnp
from jax import lax
from jax.experimental import pallas as pl
from jax.experimental.pallas import tpu as pltpu
from jax.experimental.pallas import tpu_sc as plsc

NUM_ENT = 10000
NUM_REL = 200
E = 320000
D = 128
NF = 200
KS = 7
K_W = 8
K_H = 16
OWOH = 100
FLAT = NF * OWOH
FLATP = 20480
B = 1024

NPAD = 10240
NTILE = 16
BLK = 32
BLKD = 128
EPT = 20096
EPD = NTILE * EPT
ROWS_PT = NPAD // NTILE

_F32 = jnp.float32
_HI = lax.Precision.DEFAULT


def _sc_compiler_params():
    cp = pltpu.CompilerParams()
    if "needs_layout_passes" in pltpu.CompilerParams.__dataclass_fields__:
        cp = dataclasses.replace(cp, needs_layout_passes=False)
    return cp


def _sc_deg(rows):
    mesh = plsc.VectorSubcoreMesh(core_axis_name="c", subcore_axis_name="s")
    nblk = EPT // BLKD

    @functools.partial(
        pl.kernel,
        out_type=jax.ShapeDtypeStruct((2 * NPAD,), _F32),
        mesh=mesh,
        compiler_params=_sc_compiler_params(),
        scratch_types=[
            pltpu.VMEM((BLKD,), jnp.int32),
            pltpu.VMEM((BLKD,), _F32),
            pltpu.VMEM((ROWS_PT,), _F32),
            pltpu.VMEM_SHARED((NPAD,), _F32),
        ],
    )
    def k(rows_hbm, deg_hbm, row_b, ones_b, z_b, shared_deg):
        c = lax.axis_index("c")
        s = lax.axis_index("s")
        base = c * EPD + s * EPT
        z16 = jnp.zeros((16,), _F32)

        @pl.loop(0, BLKD, step=16)
        def _(i):
            ones_b[pl.ds(i, 16)] = jnp.full((16,), 1.0, _F32)

        @pl.loop(0, ROWS_PT, step=16)
        def _(i):
            z_b[pl.ds(i, 16)] = z16

        pltpu.sync_copy(z_b, shared_deg.at[pl.ds(s * ROWS_PT, ROWS_PT)])
        plsc.subcore_barrier()

        @pl.loop(0, nblk)
        def _(b):
            pltpu.sync_copy(rows_hbm.at[pl.ds(base + b * BLKD, BLKD)], row_b)
            pltpu.sync_copy(ones_b, shared_deg.at[row_b], add=True)

        plsc.subcore_barrier()
        pltpu.sync_copy(shared_deg.at[pl.ds(s * ROWS_PT, ROWS_PT)],
                        deg_hbm.at[pl.ds(c * NPAD + s * ROWS_PT, ROWS_PT)])

    return k(rows)


def _sc_msg(rows, cols, ets, ee, xs, rels):
    mesh = plsc.VectorSubcoreMesh(core_axis_name="c", subcore_axis_name="s")
    nblk = EPT // BLK

    nsets = 2

    @functools.partial(
        pl.kernel,
        out_type=jax.ShapeDtypeStruct((2 * NPAD, D), _F32),
        mesh=mesh,
        compiler_params=_sc_compiler_params(),
        scratch_types=(
            [pltpu.VMEM((BLK,), jnp.int32) for _ in range(3 * nsets)]
            + [pltpu.VMEM((BLK,), jnp.int32) for _ in range(nsets)]
            + [pltpu.VMEM((BLK, D), _F32) for _ in range(4 * nsets)]
            + [pltpu.VMEM_SHARED((NPAD, D), _F32)]
            + [pltpu.SemaphoreType.DMA for _ in range(3 * nsets)]
        ),
    )
    def k(rows_hbm, cols_hbm, ets_hbm, ee_hbm, xs_hbm, rels_hbm, agg_hbm,
          row0, col0, et0, row1, col1, et1, colsc0, colsc1,
          xg0, rg0, eg0, xm0, xg1, rg1, eg1, xm1,
          shared_agg,
          sem_i0, sem_i1, sem_g0, sem_g1, sem_s0, sem_s1):
        c = lax.axis_index("c")
        s = lax.axis_index("s")
        base = c * EPD + s * EPT
        sets = ((row0, col0, et0, colsc0, xg0, rg0, eg0, xm0, sem_i0, sem_g0, sem_s0),
                (row1, col1, et1, colsc1, xg1, rg1, eg1, xm1, sem_i1, sem_g1, sem_s1))
        z16 = jnp.zeros((16,), _F32)

        @pl.loop(0, BLK)
        def _(r):
            for dch in range(D // 16):
                xm0[r, pl.ds(dch * 16, 16)] = z16

        for kk in range(ROWS_PT // BLK):
            pltpu.sync_copy(xm0, shared_agg.at[pl.ds(s * ROWS_PT + kk * BLK, BLK)])
        plsc.subcore_barrier()

        def idx_start(q, b):
            row_b, col_b, et_b, _, _, _, _, _, sem_i, _, _ = sets[q]
            off = base + jnp.minimum(b, nblk - 1) * BLK
            pltpu.async_copy(rows_hbm.at[pl.ds(off, BLK)], row_b, sem_i)
            pltpu.async_copy(cols_hbm.at[pl.ds(off, BLK)], col_b, sem_i)
            pltpu.async_copy(ets_hbm.at[pl.ds(off, BLK)], et_b, sem_i)

        def idx_wait(q, b):
            row_b, col_b, et_b, _, _, _, _, _, sem_i, _, _ = sets[q]
            off = base + jnp.minimum(b, nblk - 1) * BLK
            pltpu.make_async_copy(rows_hbm.at[pl.ds(off, BLK)], row_b, sem_i).wait()
            pltpu.make_async_copy(cols_hbm.at[pl.ds(off, BLK)], col_b, sem_i).wait()
            pltpu.make_async_copy(ets_hbm.at[pl.ds(off, BLK)], et_b, sem_i).wait()

        def gather_start(q, b):
            row_b, _, et_b, _, xg, rg, eg, _, _, sem_g, _ = sets[q]
            pltpu.async_copy(xs_hbm.at[row_b], xg, sem_g)
            pltpu.async_copy(rels_hbm.at[et_b], rg, sem_g)
            eoff = c * E + jnp.minimum(s * EPT + b * BLK, E - BLK)
            pltpu.async_copy(ee_hbm.at[pl.ds(eoff, BLK)], eg, sem_g)

        def gather_wait(q, b):
            row_b, _, et_b, _, xg, rg, eg, _, _, sem_g, _ = sets[q]
            pltpu.make_async_copy(xs_hbm.at[row_b], xg, sem_g).wait()
            pltpu.make_async_copy(rels_hbm.at[et_b], rg, sem_g).wait()
            eoff = c * E + jnp.minimum(s * EPT + b * BLK, E - BLK)
            pltpu.make_async_copy(ee_hbm.at[pl.ds(eoff, BLK)], eg, sem_g).wait()

        def colsc_save(q):
            _, col_b, _, colsc, _, _, _, _, _, _, _ = sets[q]
            @pl.loop(0, BLK, step=16)
            def _(i):
                colsc[pl.ds(i, 16)] = col_b[pl.ds(i, 16)]

        def scatter_start(q):
            _, _, _, colsc, _, _, _, xm, _, _, sem_s = sets[q]
            pltpu.async_copy(xm, shared_agg.at[colsc], sem_s, add=True)

        def scatter_wait(q):
            _, _, _, colsc, _, _, _, xm, _, _, sem_s = sets[q]
            pltpu.make_async_copy(xm, shared_agg.at[colsc], sem_s).wait()

        def compute(q):
            _, _, _, _, xg, rg, eg, xm, _, _, _ = sets[q]
            @functools.partial(plsc.parallel_loop, 0, BLK, unroll=4)
            def _(r):
                for dch in range(D // 16):
                    sl = pl.ds(dch * 16, 16)
                    xm[r, sl] = xg[r, sl] * rg[r, sl] * eg[r, sl]

        for q in range(2):
            idx_start(q, q)
            idx_wait(q, q)
            gather_start(q, q)

        @pl.loop(0, nblk // 2)
        def _(bb):
            b0 = 2 * bb
            for q in range(2):
                b = b0 + q

                @pl.when(bb > 0)
                def _():
                    scatter_wait(q)
                colsc_save(q)
                gather_wait(q, b)
                idx_start(q, b + 2)
                compute(q)
                scatter_start(q)
                idx_wait(q, b + 2)

                @pl.when(b + 2 < nblk)
                def _():
                    gather_start(q, b + 2)

        scatter_wait(0)
        scatter_wait(1)

        plsc.subcore_barrier()
        for kk in range(ROWS_PT // BLK):
            r0 = s * ROWS_PT + kk * BLK
            pltpu.sync_copy(shared_agg.at[pl.ds(r0, BLK)],
                            agg_hbm.at[pl.ds(c * NPAD + r0, BLK)])

    return k(rows, cols, ets, ee, xs, rels)


def _sc_gather(all_ent, all_rel, src, rel):
    mesh = plsc.VectorSubcoreMesh(core_axis_name="c", subcore_axis_name="s")
    rows_pw = B // 32

    @functools.partial(
        pl.kernel,
        out_type=(jax.ShapeDtypeStruct((B, D), _F32),
                  jax.ShapeDtypeStruct((B, D), _F32)),
        mesh=mesh,
        scratch_types=[
            pltpu.VMEM((rows_pw,), jnp.int32),
            pltpu.VMEM((rows_pw, D), _F32),
            pltpu.SemaphoreType.DMA,
        ],
    )
    def k(ae_hbm, ar_hbm, src_hbm, rel_hbm, se_hbm, re_hbm, idx_v, rows_v, sem):
        c = lax.axis_index("c")
        s = lax.axis_index("s")
        wid = s * 2 + c
        b0 = wid * rows_pw
        pltpu.sync_copy(src_hbm.at[pl.ds(b0, rows_pw)], idx_v)
        pltpu.async_copy(ae_hbm.at[idx_v], rows_v, sem).wait()
        pltpu.sync_copy(rows_v, se_hbm.at[pl.ds(b0, rows_pw)])
        pltpu.sync_copy(rel_hbm.at[pl.ds(b0, rows_pw)], idx_v)
        pltpu.async_copy(ar_hbm.at[idx_v], rows_v, sem).wait()
        pltpu.sync_copy(rows_v, re_hbm.at[pl.ds(b0, rows_pw)])

    return k(all_ent, all_rel, src, rel)


def _tc_dinvscale(deg2, xp):
    nb = 2 * NPAD // 1024

    def body(deg_ref, x_ref, xs_ref, dv_ref):
        dg = deg_ref[...]
        dv = jnp.where(dg > 0.5, lax.rsqrt(jnp.maximum(dg, 1e-12)), 0.0)
        dv_ref[...] = dv
        xs_ref[...] = x_ref[...] * dv

    return pl.pallas_call(
        body,
        grid=(nb,),
        in_specs=[pl.BlockSpec((1024, 1), lambda i: (i, 0)),
                  pl.BlockSpec((1024, D), lambda i: (i % (NPAD // 1024), 0))],
        out_specs=[pl.BlockSpec((1024, D), lambda i: (i, 0)),
                   pl.BlockSpec((1024, 1), lambda i: (i, 0))],
        out_shape=[jax.ShapeDtypeStruct((2 * NPAD, D), _F32),
                   jax.ShapeDtypeStruct((2 * NPAD, 1), _F32)],
    )(deg2, xp)


def _tc_encoder(agg_in, agg_out, dv0, dv1, xp, coeff, wi, wo, wl):
    nb = NPAD // 1024

    def body(ai_ref, ao_ref, d0_ref, d1_ref, x_ref, cf_ref, wi_ref, wo_ref,
             wl_ref, pre_ref, st_ref):
        i = pl.program_id(0)
        xc = x_ref[...] * cf_ref[...]
        pre = (jnp.dot(ai_ref[...] * d0_ref[...], wi_ref[...],
                       preferred_element_type=_F32, precision=_HI)
               + jnp.dot(ao_ref[...] * d1_ref[...], wo_ref[...],
                         preferred_element_type=_F32, precision=_HI)
               + jnp.dot(xc, wl_ref[...], preferred_element_type=_F32,
                         precision=_HI)) * (1.0 / 3.0)
        pre_ref[...] = pre

        @pl.when(i == 0)
        def _():
            st_ref[...] = jnp.zeros_like(st_ref)

        st_ref[0:1, :] += jnp.sum(pre, axis=0, keepdims=True)
        st_ref[1:2, :] += jnp.sum(pre * pre, axis=0, keepdims=True)

    blk = pl.BlockSpec((1024, D), lambda i: (i, 0))
    dblk = pl.BlockSpec((1024, 1), lambda i: (i, 0))
    wblk = pl.BlockSpec((D, D), lambda i: (0, 0))
    return pl.pallas_call(
        body,
        grid=(nb,),
        in_specs=[blk, blk, dblk, dblk, blk, pl.BlockSpec((1, D), lambda i: (0, 0)),
                  wblk, wblk, wblk],
        out_specs=[blk, pl.BlockSpec((8, D), lambda i: (0, 0))],
        out_shape=[jax.ShapeDtypeStruct((NPAD, D), _F32),
                   jax.ShapeDtypeStruct((8, D), _F32)],
    )(agg_in, agg_out, dv0, dv1, xp, coeff, wi, wo, wl)


def _tc_entnorm(pre, bnpack):
    nb = NPAD // 1024

    def body(pre_ref, bn_ref, ae_ref):
        mean = bn_ref[0:1, :]
        gs = bn_ref[1:2, :]
        bb = bn_ref[2:3, :]
        ae_ref[...] = jnp.tanh((pre_ref[...] - mean) * gs + bb)

    blk = pl.BlockSpec((1024, D), lambda i: (i, 0))
    return pl.pallas_call(
        body,
        grid=(nb,),
        in_specs=[blk, pl.BlockSpec((8, D), lambda i: (0, 0))],
        out_specs=blk,
        out_shape=jax.ShapeDtypeStruct((NPAD, D), _F32),
    )(pre, bnpack)


def _tc_relmm(rels_pad, wr):
    def body(r_ref, w_ref, o_ref):
        o_ref[...] = jnp.dot(r_ref[...], w_ref[...], preferred_element_type=_F32,
                             precision=_HI)

    return pl.pallas_call(
        body,
        out_shape=jax.ShapeDtypeStruct((512, D), _F32),
    )(rels_pad, wr)


def _tc_convmm(cat, wc):
    nb = FLATP // 128

    def body(cat_ref, wc_ref, raw_ref, st_ref, cst_ref):
        j = pl.program_id(0)
        catv = cat_ref[...]
        raw = jnp.dot(catv, wc_ref[...], preferred_element_type=_F32,
                      precision=_HI)
        raw_ref[...] = raw
        st_ref[...] = jnp.concatenate(
            [jnp.sum(raw, axis=0, keepdims=True),
             jnp.sum(raw * raw, axis=0, keepdims=True),
             jnp.zeros((6, 128), _F32)], axis=0)

        @pl.when(j == 0)
        def _():
            cst_ref[...] = jnp.concatenate(
                [jnp.sum(catv, axis=0, keepdims=True),
                 jnp.sum(catv * catv, axis=0, keepdims=True),
                 jnp.zeros((6, 2 * D), _F32)], axis=0)

    return pl.pallas_call(
        body,
        grid=(nb,),
        in_specs=[pl.BlockSpec((B, 2 * D), lambda j: (0, 0)),
                  pl.BlockSpec((2 * D, 128), lambda j: (0, j))],
        out_specs=[pl.BlockSpec((B, 128), lambda j: (0, j)),
                   pl.BlockSpec((8, 128), lambda j: (0, j)),
                   pl.BlockSpec((8, 2 * D), lambda j: (0, 0))],
        out_shape=[jax.ShapeDtypeStruct((B, FLATP), _F32),
                   jax.ShapeDtypeStruct((8, FLATP), _F32),
                   jax.ShapeDtypeStruct((8, 2 * D), _F32)],
    )(cat, wc)


def _tc_fc(raw, ab, fcw2, fcb):
    nb = FLATP // 128

    def body(raw_ref, ab_ref, w_ref, fcb_ref, h_ref, st_ref):
        j = pl.program_id(0)
        h1 = jnp.maximum(raw_ref[...] * ab_ref[0:1, :] + ab_ref[1:2, :], 0.0)

        @pl.when(j == 0)
        def _():
            h_ref[...] = jnp.broadcast_to(fcb_ref[...], (B, D))

        h_ref[...] += jnp.dot(h1, w_ref[...], preferred_element_type=_F32,
                              precision=_HI)

        @pl.when(j == nb - 1)
        def _():
            h = h_ref[...]
            st_ref[...] = jnp.concatenate(
                [jnp.sum(h, axis=0, keepdims=True),
                 jnp.sum(h * h, axis=0, keepdims=True),
                 jnp.zeros((6, D), _F32)], axis=0)

    return pl.pallas_call(
        body,
        grid=(nb,),
        in_specs=[pl.BlockSpec((B, 128), lambda j: (0, j)),
                  pl.BlockSpec((8, 128), lambda j: (0, j)),
                  pl.BlockSpec((128, D), lambda j: (j, 0)),
                  pl.BlockSpec((1, D), lambda j: (0, 0))],
        out_specs=[pl.BlockSpec((B, D), lambda j: (0, 0)),
                   pl.BlockSpec((8, D), lambda j: (0, 0))],
        out_shape=[jax.ShapeDtypeStruct((B, D), _F32),
                   jax.ShapeDtypeStruct((8, D), _F32)],
    )(raw, ab, fcw2, fcb)


def _tc_score(h, bn2pack, all_ent, bias):
    nb = NPAD // 1024

    def body(h_ref, bn_ref, ae_ref, bias_ref, o_ref):
        h2 = jnp.maximum((h_ref[...] - bn_ref[0:1, :]) * bn_ref[1:2, :]
                         + bn_ref[2:3, :], 0.0)
        logits = lax.dot_general(h2, ae_ref[...], (((1,), (1,)), ((), ())),
                                 preferred_element_type=_F32, precision=_HI)
        o_ref[...] = jax.nn.sigmoid(logits + bias_ref[...])

    return pl.pallas_call(
        body,
        grid=(nb,),
        in_specs=[pl.BlockSpec((B, D), lambda j: (0, 0)),
                  pl.BlockSpec((8, D), lambda j: (0, 0)),
                  pl.BlockSpec((1024, D), lambda j: (j, 0)),
                  pl.BlockSpec((1, 1024), lambda j: (0, j))],
        out_specs=pl.BlockSpec((B, 1024), lambda j: (0, j)),
        out_shape=jax.ShapeDtypeStruct((B, NPAD), _F32),
    )(h, bn2pack, all_ent, bias)


def _conv_qidx():
    qidx = np.full((2 * D, OWOH), KS * KS, np.int32)
    for rowcat in range(2 * D):
        c, dd = rowcat // D, rowcat % D
        p = 2 * dd + c
        pi, pj = p // K_H, p % K_H
        for ij in range(OWOH):
            i, j = ij // 10, ij % 10
            ki, kj = pi - i, pj - j
            if 0 <= ki < KS and 0 <= kj < KS:
                qidx[rowcat, ij] = ki * KS + kj
    return qidx.reshape(-1)

_QIDX = _conv_qidx()


def kernel(src, rel, entity, edge_index, edge_norm, edge_type, edge_ids,
           entity_embedding, relation_embedding, edge_embeddings,
           in_weight, out_weight, loop_weight, rels_weight, loop_rel, loop_edge,
           ent_bn_g, ent_bn_b, bn0_g, bn0_b, bn1_g, bn1_b, bn2_g, bn2_b,
           conv_w, fc_w, fc_b, ent_bias):
    f32 = _F32
    half = E

    ei = edge_index.astype(jnp.int32)
    et = edge_type.astype(jnp.int32)

    def _pad_edges(v, fill):
        return jnp.pad(v, (0, EPD - E), constant_values=fill)

    rows_in = _pad_edges(ei[0, :half], NPAD - 1)
    rows_out = _pad_edges(ei[0, half:], NPAD - 1)
    rows = jnp.concatenate([rows_in, rows_out])
    rows_off = jnp.concatenate([rows_in, rows_out + NPAD])
    cols = jnp.concatenate([_pad_edges(ei[1, :half], NPAD - 1),
                            _pad_edges(ei[1, half:], NPAD - 1)])
    ets = jnp.concatenate([_pad_edges(et[:half], 0), _pad_edges(et[half:], 0)])

    xp = jnp.pad(entity_embedding.astype(f32), ((0, NPAD - NUM_ENT), (0, 0)))
    rels = jnp.concatenate([relation_embedding, loop_rel], axis=0).astype(f32)
    rels_pad = jnp.pad(rels, ((0, 512 - rels.shape[0]), (0, 0)))

    deg = _sc_deg(rows)
    xs, dvc = _tc_dinvscale(deg.reshape(2 * NPAD, 1), xp)

    agg = _sc_msg(rows_off, cols, ets, edge_embeddings.astype(f32), xs, rels_pad)
    agg_in, agg_out = agg[:NPAD], agg[NPAD:]

    coeff = (loop_rel * loop_edge).astype(f32)
    pre, st = _tc_encoder(agg_in, agg_out, dvc[:NPAD], dvc[NPAD:], xp, coeff,
                          in_weight.astype(f32), out_weight.astype(f32),
                          loop_weight.astype(f32))
    m = st[0] / NUM_ENT
    v = st[1] / NUM_ENT - m * m
    gs = ent_bn_g / jnp.sqrt(v + 1e-5)
    bnpack = jnp.zeros((8, D), f32).at[0].set(m).at[1].set(gs).at[2].set(ent_bn_b)
    all_ent = _tc_entnorm(pre, bnpack)

    all_rel = _tc_relmm(rels_pad, rels_weight.astype(f32))

    src_emb, rel_emb = _sc_gather(all_ent, all_rel, src.astype(jnp.int32),
                                  rel.astype(jnp.int32))
    cat = jnp.concatenate([src_emb, rel_emb], axis=1)

    cwT = conv_w.reshape(NF, KS * KS).T.astype(f32)
    cwT_ext = jnp.concatenate([cwT, jnp.zeros((1, NF), f32)], axis=0)
    wc = jnp.take(cwT_ext, jnp.asarray(_QIDX), axis=0).reshape(2 * D, FLAT)
    wc = jnp.pad(wc, ((0, 0), (0, FLATP - FLAT)))
    wsum = jnp.sum(wc, axis=0)

    raw, rst, cst = _tc_convmm(cat, wc)

    n0 = B * 2 * D
    s0 = jnp.sum(cst[0])
    s0sq = jnp.sum(cst[1])
    m0 = s0 / n0
    v0 = s0sq / n0 - m0 * m0
    a0 = bn0_g[0] / jnp.sqrt(v0 + 1e-5)
    c0 = bn0_b[0] - m0 * a0

    csum, csumsq = rst[0], rst[1]
    co_sum = a0 * csum + B * c0 * wsum
    co_sumsq = (a0 * a0 * csumsq + 2 * a0 * c0 * wsum * csum
                + B * (c0 * wsum) ** 2)
    g_sum = co_sum[:FLAT].reshape(OWOH, NF).sum(0)
    g_sumsq = co_sumsq[:FLAT].reshape(OWOH, NF).sum(0)
    n1 = B * OWOH
    m1 = g_sum / n1
    v1 = g_sumsq / n1 - m1 * m1
    alpha_f = bn1_g / jnp.sqrt(v1 + 1e-5)
    beta_f = bn1_b - m1 * alpha_f
    alpha = jnp.pad(jnp.tile(alpha_f * a0, OWOH), (0, FLATP - FLAT))
    beta = jnp.pad(jnp.tile(alpha_f, OWOH) * c0 * wsum[:FLAT]
                   + jnp.tile(beta_f, OWOH), (0, FLATP - FLAT))
    ab = jnp.zeros((8, FLATP), f32).at[0].set(alpha).at[1].set(beta)

    fc_w2 = fc_w.reshape(NF, OWOH, D).transpose(1, 0, 2).reshape(FLAT, D)
    fc_w2 = jnp.pad(fc_w2.astype(f32), ((0, FLATP - FLAT), (0, 0)))

    h, hst = _tc_fc(raw, ab, fc_w2, fc_b.reshape(1, D).astype(f32))

    m2 = hst[0] / B
    v2 = hst[1] / B - m2 * m2
    g2s = bn2_g / jnp.sqrt(v2 + 1e-5)
    bn2pack = jnp.zeros((8, D), f32).at[0].set(m2).at[1].set(g2s).at[2].set(bn2_b)

    bias = jnp.pad(ent_bias.reshape(1, NUM_ENT).astype(f32),
                   ((0, 0), (0, NPAD - NUM_ENT)))
    score = _tc_score(h, bn2pack, all_ent, bias)
    return score[:, :NUM_ENT]


# speedup vs baseline: 1.1173x; 1.0377x over previous
import dataclasses
import functools

import numpy as np
import jax
import jax.numpy as jnp
from jax import lax
from jax.experimental import pallas as pl
from jax.experimental.pallas import tpu as pltpu
from jax.experimental.pallas import tpu_sc as plsc

NUM_ENT = 10000
NUM_REL = 200
E = 320000
D = 128
NF = 200
KS = 7
K_W = 8
K_H = 16
OWOH = 100
FLAT = NF * OWOH
FLATP = 20480
B = 1024

NPAD = 10240
NTILE = 16
BLK = 32
BLKD = 128
EPT = 20096
EPD = NTILE * EPT
ROWS_PT = NPAD // NTILE

_F32 = jnp.float32
_HI = lax.Precision.DEFAULT


def _sc_compiler_params():
    cp = pltpu.CompilerParams()
    if "needs_layout_passes" in pltpu.CompilerParams.__dataclass_fields__:
        cp = dataclasses.replace(cp, needs_layout_passes=False)
    return cp


def _sc_deg(rows):
    mesh = plsc.VectorSubcoreMesh(core_axis_name="c", subcore_axis_name="s")
    nblk = EPT // BLKD

    @functools.partial(
        pl.kernel,
        out_type=jax.ShapeDtypeStruct((2 * NPAD,), _F32),
        mesh=mesh,
        compiler_params=_sc_compiler_params(),
        scratch_types=[
            pltpu.VMEM((BLKD,), jnp.int32),
            pltpu.VMEM((BLKD,), jnp.int32),
            pltpu.VMEM((BLKD,), jnp.int32),
            pltpu.VMEM((BLKD,), jnp.int32),
            pltpu.VMEM((BLKD,), _F32),
            pltpu.VMEM((ROWS_PT,), _F32),
            pltpu.VMEM_SHARED((NPAD,), _F32),
            pltpu.SemaphoreType.DMA,
        ],
    )
    def k(rows_hbm, deg_hbm, row_b0, row_b1, row_b2, row_b3, ones_b, z_b,
          shared_deg, sem_i):
        c = lax.axis_index("c")
        s = lax.axis_index("s")
        base = c * EPD + s * EPT
        z16 = jnp.zeros((16,), _F32)

        @pl.loop(0, BLKD, step=16)
        def _(i):
            ones_b[pl.ds(i, 16)] = jnp.full((16,), 1.0, _F32)

        @pl.loop(0, ROWS_PT, step=16)
        def _(i):
            z_b[pl.ds(i, 16)] = z16

        pltpu.sync_copy(z_b, shared_deg.at[pl.ds(s * ROWS_PT, ROWS_PT)])
        plsc.subcore_barrier()

        rbs = (row_b0, row_b1, row_b2, row_b3)

        @pl.loop(0, nblk, step=4)
        def _(b):
            nq = jnp.minimum(nblk - b, 4)
            for j in range(4):
                @pl.when(j < nq)
                def _():
                    pltpu.async_copy(
                        rows_hbm.at[pl.ds(base + (b + j) * BLKD, BLKD)],
                        rbs[j], sem_i)
            for j in range(4):
                @pl.when(j < nq)
                def _():
                    pltpu.make_async_copy(
                        rows_hbm.at[pl.ds(base + (b + j) * BLKD, BLKD)],
                        rbs[j], sem_i).wait()
                    pltpu.sync_copy(ones_b, shared_deg.at[rbs[j]], add=True)

        plsc.subcore_barrier()
        pltpu.sync_copy(shared_deg.at[pl.ds(s * ROWS_PT, ROWS_PT)],
                        deg_hbm.at[pl.ds(c * NPAD + s * ROWS_PT, ROWS_PT)])

    return k(rows)


def _sc_msg(rows, cols, ets, ee, xs, rels):
    mesh = plsc.VectorSubcoreMesh(core_axis_name="c", subcore_axis_name="s")
    nblk = EPT // BLK

    nsets = 2

    @functools.partial(
        pl.kernel,
        out_type=jax.ShapeDtypeStruct((2 * NPAD, D), _F32),
        mesh=mesh,
        compiler_params=_sc_compiler_params(),
        scratch_types=(
            [pltpu.VMEM((BLK,), jnp.int32) for _ in range(3 * nsets)]
            + [pltpu.VMEM((BLK,), jnp.int32) for _ in range(nsets)]
            + [pltpu.VMEM((BLK, D), _F32) for _ in range(4 * nsets)]
            + [pltpu.VMEM_SHARED((NPAD, D), _F32)]
            + [pltpu.SemaphoreType.DMA for _ in range(3 * nsets)]
        ),
    )
    def k(rows_hbm, cols_hbm, ets_hbm, ee_hbm, xs_hbm, rels_hbm, agg_hbm,
          row0, col0, et0, row1, col1, et1, colsc0, colsc1,
          xg0, rg0, eg0, xm0, xg1, rg1, eg1, xm1,
          shared_agg,
          sem_i0, sem_i1, sem_g0, sem_g1, sem_s0, sem_s1):
        c = lax.axis_index("c")
        s = lax.axis_index("s")
        base = c * EPD + s * EPT
        sets = ((row0, col0, et0, colsc0, xg0, rg0, eg0, xm0, sem_i0, sem_g0, sem_s0),
                (row1, col1, et1, colsc1, xg1, rg1, eg1, xm1, sem_i1, sem_g1, sem_s1))
        z16 = jnp.zeros((16,), _F32)

        @pl.loop(0, BLK)
        def _(r):
            for dch in range(D // 16):
                xm0[r, pl.ds(dch * 16, 16)] = z16

        for kk in range(ROWS_PT // BLK):
            pltpu.sync_copy(xm0, shared_agg.at[pl.ds(s * ROWS_PT + kk * BLK, BLK)])
        plsc.subcore_barrier()

        def idx_start(q, b):
            row_b, col_b, et_b, _, _, _, _, _, sem_i, _, _ = sets[q]
            off = base + jnp.minimum(b, nblk - 1) * BLK
            pltpu.async_copy(rows_hbm.at[pl.ds(off, BLK)], row_b, sem_i)
            pltpu.async_copy(cols_hbm.at[pl.ds(off, BLK)], col_b, sem_i)
            pltpu.async_copy(ets_hbm.at[pl.ds(off, BLK)], et_b, sem_i)

        def idx_wait(q, b):
            row_b, col_b, et_b, _, _, _, _, _, sem_i, _, _ = sets[q]
            off = base + jnp.minimum(b, nblk - 1) * BLK
            pltpu.make_async_copy(rows_hbm.at[pl.ds(off, BLK)], row_b, sem_i).wait()
            pltpu.make_async_copy(cols_hbm.at[pl.ds(off, BLK)], col_b, sem_i).wait()
            pltpu.make_async_copy(ets_hbm.at[pl.ds(off, BLK)], et_b, sem_i).wait()

        def gather_start(q, b):
            row_b, _, et_b, _, xg, rg, eg, _, _, sem_g, _ = sets[q]
            pltpu.async_copy(xs_hbm.at[row_b], xg, sem_g)
            pltpu.async_copy(rels_hbm.at[et_b], rg, sem_g)
            eoff = c * E + jnp.minimum(s * EPT + b * BLK, E - BLK)
            pltpu.async_copy(ee_hbm.at[pl.ds(eoff, BLK)], eg, sem_g)

        def gather_wait(q, b):
            row_b, _, et_b, _, xg, rg, eg, _, _, sem_g, _ = sets[q]
            pltpu.make_async_copy(xs_hbm.at[row_b], xg, sem_g).wait()
            pltpu.make_async_copy(rels_hbm.at[et_b], rg, sem_g).wait()
            eoff = c * E + jnp.minimum(s * EPT + b * BLK, E - BLK)
            pltpu.make_async_copy(ee_hbm.at[pl.ds(eoff, BLK)], eg, sem_g).wait()

        def colsc_save(q):
            _, col_b, _, colsc, _, _, _, _, _, _, _ = sets[q]
            @pl.loop(0, BLK, step=16)
            def _(i):
                colsc[pl.ds(i, 16)] = col_b[pl.ds(i, 16)]

        def scatter_start(q):
            _, _, _, colsc, _, _, _, xm, _, _, sem_s = sets[q]
            pltpu.async_copy(xm, shared_agg.at[colsc], sem_s, add=True)

        def scatter_wait(q):
            _, _, _, colsc, _, _, _, xm, _, _, sem_s = sets[q]
            pltpu.make_async_copy(xm, shared_agg.at[colsc], sem_s).wait()

        def compute(q):
            _, _, _, _, xg, rg, eg, xm, _, _, _ = sets[q]
            @pl.loop(0, BLK)
            def _(r):
                for dch in range(D // 16):
                    sl = pl.ds(dch * 16, 16)
                    xm[r, sl] = xg[r, sl] * rg[r, sl] * eg[r, sl]

        for q in range(2):
            idx_start(q, q)
            idx_wait(q, q)
            gather_start(q, q)

        @pl.loop(0, nblk // 2)
        def _(bb):
            b0 = 2 * bb
            for q in range(2):
                b = b0 + q

                @pl.when(bb > 0)
                def _():
                    scatter_wait(q)
                colsc_save(q)
                gather_wait(q, b)
                idx_start(q, b + 2)
                compute(q)
                scatter_start(q)
                idx_wait(q, b + 2)

                @pl.when(b + 2 < nblk)
                def _():
                    gather_start(q, b + 2)

        scatter_wait(0)
        scatter_wait(1)

        plsc.subcore_barrier()
        for kk in range(ROWS_PT // BLK):
            r0 = s * ROWS_PT + kk * BLK
            pltpu.sync_copy(shared_agg.at[pl.ds(r0, BLK)],
                            agg_hbm.at[pl.ds(c * NPAD + r0, BLK)])

    return k(rows, cols, ets, ee, xs, rels)


def _sc_gather(all_ent, all_rel, src, rel):
    mesh = plsc.VectorSubcoreMesh(core_axis_name="c", subcore_axis_name="s")
    rows_pw = B // 32

    @functools.partial(
        pl.kernel,
        out_type=(jax.ShapeDtypeStruct((B, D), _F32),
                  jax.ShapeDtypeStruct((B, D), _F32)),
        mesh=mesh,
        scratch_types=[
            pltpu.VMEM((rows_pw,), jnp.int32),
            pltpu.VMEM((rows_pw, D), _F32),
            pltpu.SemaphoreType.DMA,
        ],
    )
    def k(ae_hbm, ar_hbm, src_hbm, rel_hbm, se_hbm, re_hbm, idx_v, rows_v, sem):
        c = lax.axis_index("c")
        s = lax.axis_index("s")
        wid = s * 2 + c
        b0 = wid * rows_pw
        pltpu.sync_copy(src_hbm.at[pl.ds(b0, rows_pw)], idx_v)
        pltpu.async_copy(ae_hbm.at[idx_v], rows_v, sem).wait()
        pltpu.sync_copy(rows_v, se_hbm.at[pl.ds(b0, rows_pw)])
        pltpu.sync_copy(rel_hbm.at[pl.ds(b0, rows_pw)], idx_v)
        pltpu.async_copy(ar_hbm.at[idx_v], rows_v, sem).wait()
        pltpu.sync_copy(rows_v, re_hbm.at[pl.ds(b0, rows_pw)])

    return k(all_ent, all_rel, src, rel)


def _tc_dinvscale(deg2, xp):
    nb = 2 * NPAD // 1024

    def body(deg_ref, x_ref, xs_ref, dv_ref):
        dg = deg_ref[...]
        dv = jnp.where(dg > 0.5, lax.rsqrt(jnp.maximum(dg, 1e-12)), 0.0)
        dv_ref[...] = dv
        xs_ref[...] = x_ref[...] * dv

    return pl.pallas_call(
        body,
        grid=(nb,),
        in_specs=[pl.BlockSpec((1024, 1), lambda i: (i, 0)),
                  pl.BlockSpec((1024, D), lambda i: (i % (NPAD // 1024), 0))],
        out_specs=[pl.BlockSpec((1024, D), lambda i: (i, 0)),
                   pl.BlockSpec((1024, 1), lambda i: (i, 0))],
        out_shape=[jax.ShapeDtypeStruct((2 * NPAD, D), _F32),
                   jax.ShapeDtypeStruct((2 * NPAD, 1), _F32)],
    )(deg2, xp)


def _tc_encoder(agg_in, agg_out, dv0, dv1, xp, coeff, wi, wo, wl):
    nb = NPAD // 1024

    def body(ai_ref, ao_ref, d0_ref, d1_ref, x_ref, cf_ref, wi_ref, wo_ref,
             wl_ref, pre_ref, st_ref):
        i = pl.program_id(0)
        xc = x_ref[...] * cf_ref[...]
        pre = (jnp.dot(ai_ref[...] * d0_ref[...], wi_ref[...],
                       preferred_element_type=_F32, precision=_HI)
               + jnp.dot(ao_ref[...] * d1_ref[...], wo_ref[...],
                         preferred_element_type=_F32, precision=_HI)
               + jnp.dot(xc, wl_ref[...], preferred_element_type=_F32,
                         precision=_HI)) * (1.0 / 3.0)
        pre_ref[...] = pre

        @pl.when(i == 0)
        def _():
            st_ref[...] = jnp.zeros_like(st_ref)

        st_ref[0:1, :] += jnp.sum(pre, axis=0, keepdims=True)
        st_ref[1:2, :] += jnp.sum(pre * pre, axis=0, keepdims=True)

    blk = pl.BlockSpec((1024, D), lambda i: (i, 0))
    dblk = pl.BlockSpec((1024, 1), lambda i: (i, 0))
    wblk = pl.BlockSpec((D, D), lambda i: (0, 0))
    return pl.pallas_call(
        body,
        grid=(nb,),
        in_specs=[blk, blk, dblk, dblk, blk, pl.BlockSpec((1, D), lambda i: (0, 0)),
                  wblk, wblk, wblk],
        out_specs=[blk, pl.BlockSpec((8, D), lambda i: (0, 0))],
        out_shape=[jax.ShapeDtypeStruct((NPAD, D), _F32),
                   jax.ShapeDtypeStruct((8, D), _F32)],
    )(agg_in, agg_out, dv0, dv1, xp, coeff, wi, wo, wl)


def _tc_entnorm(pre, bnpack):
    nb = NPAD // 1024

    def body(pre_ref, bn_ref, ae_ref):
        mean = bn_ref[0:1, :]
        gs = bn_ref[1:2, :]
        bb = bn_ref[2:3, :]
        ae_ref[...] = jnp.tanh((pre_ref[...] - mean) * gs + bb)

    blk = pl.BlockSpec((1024, D), lambda i: (i, 0))
    return pl.pallas_call(
        body,
        grid=(nb,),
        in_specs=[blk, pl.BlockSpec((8, D), lambda i: (0, 0))],
        out_specs=blk,
        out_shape=jax.ShapeDtypeStruct((NPAD, D), _F32),
    )(pre, bnpack)


def _tc_relmm(rels_pad, wr):
    def body(r_ref, w_ref, o_ref):
        o_ref[...] = jnp.dot(r_ref[...], w_ref[...], preferred_element_type=_F32,
                             precision=_HI)

    return pl.pallas_call(
        body,
        out_shape=jax.ShapeDtypeStruct((512, D), _F32),
    )(rels_pad, wr)


def _tc_convmm(cat, wc):
    nb = FLATP // 128

    def body(cat_ref, wc_ref, raw_ref, st_ref, cst_ref):
        j = pl.program_id(0)
        catv = cat_ref[...]
        raw = jnp.dot(catv, wc_ref[...], preferred_element_type=_F32,
                      precision=_HI)
        raw_ref[...] = raw
        st_ref[...] = jnp.concatenate(
            [jnp.sum(raw, axis=0, keepdims=True),
             jnp.sum(raw * raw, axis=0, keepdims=True),
             jnp.zeros((6, 128), _F32)], axis=0)

        @pl.when(j == 0)
        def _():
            cst_ref[...] = jnp.concatenate(
                [jnp.sum(catv, axis=0, keepdims=True),
                 jnp.sum(catv * catv, axis=0, keepdims=True),
                 jnp.zeros((6, 2 * D), _F32)], axis=0)

    return pl.pallas_call(
        body,
        grid=(nb,),
        in_specs=[pl.BlockSpec((B, 2 * D), lambda j: (0, 0)),
                  pl.BlockSpec((2 * D, 128), lambda j: (0, j))],
        out_specs=[pl.BlockSpec((B, 128), lambda j: (0, j)),
                   pl.BlockSpec((8, 128), lambda j: (0, j)),
                   pl.BlockSpec((8, 2 * D), lambda j: (0, 0))],
        out_shape=[jax.ShapeDtypeStruct((B, FLATP), _F32),
                   jax.ShapeDtypeStruct((8, FLATP), _F32),
                   jax.ShapeDtypeStruct((8, 2 * D), _F32)],
    )(cat, wc)


def _tc_fc(raw, ab, fcw2, fcb):
    nb = FLATP // 128

    def body(raw_ref, ab_ref, w_ref, fcb_ref, h_ref, st_ref):
        j = pl.program_id(0)
        h1 = jnp.maximum(raw_ref[...] * ab_ref[0:1, :] + ab_ref[1:2, :], 0.0)

        @pl.when(j == 0)
        def _():
            h_ref[...] = jnp.broadcast_to(fcb_ref[...], (B, D))

        h_ref[...] += jnp.dot(h1, w_ref[...], preferred_element_type=_F32,
                              precision=_HI)

        @pl.when(j == nb - 1)
        def _():
            h = h_ref[...]
            st_ref[...] = jnp.concatenate(
                [jnp.sum(h, axis=0, keepdims=True),
                 jnp.sum(h * h, axis=0, keepdims=True),
                 jnp.zeros((6, D), _F32)], axis=0)

    return pl.pallas_call(
        body,
        grid=(nb,),
        in_specs=[pl.BlockSpec((B, 128), lambda j: (0, j)),
                  pl.BlockSpec((8, 128), lambda j: (0, j)),
                  pl.BlockSpec((128, D), lambda j: (j, 0)),
                  pl.BlockSpec((1, D), lambda j: (0, 0))],
        out_specs=[pl.BlockSpec((B, D), lambda j: (0, 0)),
                   pl.BlockSpec((8, D), lambda j: (0, 0))],
        out_shape=[jax.ShapeDtypeStruct((B, D), _F32),
                   jax.ShapeDtypeStruct((8, D), _F32)],
    )(raw, ab, fcw2, fcb)


def _tc_score(h, bn2pack, all_ent, bias):
    nb = NPAD // 1024

    def body(h_ref, bn_ref, ae_ref, bias_ref, o_ref):
        h2 = jnp.maximum((h_ref[...] - bn_ref[0:1, :]) * bn_ref[1:2, :]
                         + bn_ref[2:3, :], 0.0)
        logits = lax.dot_general(h2, ae_ref[...], (((1,), (1,)), ((), ())),
                                 preferred_element_type=_F32, precision=_HI)
        o_ref[...] = jax.nn.sigmoid(logits + bias_ref[...])

    return pl.pallas_call(
        body,
        grid=(nb,),
        in_specs=[pl.BlockSpec((B, D), lambda j: (0, 0)),
                  pl.BlockSpec((8, D), lambda j: (0, 0)),
                  pl.BlockSpec((1024, D), lambda j: (j, 0)),
                  pl.BlockSpec((1, 1024), lambda j: (0, j))],
        out_specs=pl.BlockSpec((B, 1024), lambda j: (0, j)),
        out_shape=jax.ShapeDtypeStruct((B, NPAD), _F32),
    )(h, bn2pack, all_ent, bias)


def _conv_qidx():
    qidx = np.full((2 * D, OWOH), KS * KS, np.int32)
    for rowcat in range(2 * D):
        c, dd = rowcat // D, rowcat % D
        p = 2 * dd + c
        pi, pj = p // K_H, p % K_H
        for ij in range(OWOH):
            i, j = ij // 10, ij % 10
            ki, kj = pi - i, pj - j
            if 0 <= ki < KS and 0 <= kj < KS:
                qidx[rowcat, ij] = ki * KS + kj
    return qidx.reshape(-1)

_QIDX = _conv_qidx()


def kernel(src, rel, entity, edge_index, edge_norm, edge_type, edge_ids,
           entity_embedding, relation_embedding, edge_embeddings,
           in_weight, out_weight, loop_weight, rels_weight, loop_rel, loop_edge,
           ent_bn_g, ent_bn_b, bn0_g, bn0_b, bn1_g, bn1_b, bn2_g, bn2_b,
           conv_w, fc_w, fc_b, ent_bias):
    f32 = _F32
    half = E

    ei = edge_index.astype(jnp.int32)
    et = edge_type.astype(jnp.int32)

    def _pad_edges(v, fill):
        return jnp.pad(v, (0, EPD - E), constant_values=fill)

    rows_in = _pad_edges(ei[0, :half], NPAD - 1)
    rows_out = _pad_edges(ei[0, half:], NPAD - 1)
    rows = jnp.concatenate([rows_in, rows_out])
    rows_off = jnp.concatenate([rows_in, rows_out + NPAD])
    cols = jnp.concatenate([_pad_edges(ei[1, :half], NPAD - 1),
                            _pad_edges(ei[1, half:], NPAD - 1)])
    ets = jnp.concatenate([_pad_edges(et[:half], 0), _pad_edges(et[half:], 0)])

    xp = jnp.pad(entity_embedding.astype(f32), ((0, NPAD - NUM_ENT), (0, 0)))
    rels = jnp.concatenate([relation_embedding, loop_rel], axis=0).astype(f32)
    rels_pad = jnp.pad(rels, ((0, 512 - rels.shape[0]), (0, 0)))

    deg = _sc_deg(rows)
    xs, dvc = _tc_dinvscale(deg.reshape(2 * NPAD, 1), xp)

    agg = _sc_msg(rows_off, cols, ets, edge_embeddings.astype(f32), xs, rels_pad)
    agg_in, agg_out = agg[:NPAD], agg[NPAD:]

    coeff = (loop_rel * loop_edge).astype(f32)
    pre, st = _tc_encoder(agg_in, agg_out, dvc[:NPAD], dvc[NPAD:], xp, coeff,
                          in_weight.astype(f32), out_weight.astype(f32),
                          loop_weight.astype(f32))
    m = st[0] / NUM_ENT
    v = st[1] / NUM_ENT - m * m
    gs = ent_bn_g / jnp.sqrt(v + 1e-5)
    bnpack = jnp.zeros((8, D), f32).at[0].set(m).at[1].set(gs).at[2].set(ent_bn_b)
    all_ent = _tc_entnorm(pre, bnpack)

    all_rel = _tc_relmm(rels_pad, rels_weight.astype(f32))

    src_emb, rel_emb = _sc_gather(all_ent, all_rel, src.astype(jnp.int32),
                                  rel.astype(jnp.int32))
    cat = jnp.concatenate([src_emb, rel_emb], axis=1)

    cwT = conv_w.reshape(NF, KS * KS).T.astype(f32)
    cwT_ext = jnp.concatenate([cwT, jnp.zeros((1, NF), f32)], axis=0)
    wc = jnp.take(cwT_ext, jnp.asarray(_QIDX), axis=0).reshape(2 * D, FLAT)
    wc = jnp.pad(wc, ((0, 0), (0, FLATP - FLAT)))
    wsum = jnp.sum(wc, axis=0)

    raw, rst, cst = _tc_convmm(cat, wc)

    n0 = B * 2 * D
    s0 = jnp.sum(cst[0])
    s0sq = jnp.sum(cst[1])
    m0 = s0 / n0
    v0 = s0sq / n0 - m0 * m0
    a0 = bn0_g[0] / jnp.sqrt(v0 + 1e-5)
    c0 = bn0_b[0] - m0 * a0

    csum, csumsq = rst[0], rst[1]
    co_sum = a0 * csum + B * c0 * wsum
    co_sumsq = (a0 * a0 * csumsq + 2 * a0 * c0 * wsum * csum
                + B * (c0 * wsum) ** 2)
    g_sum = co_sum[:FLAT].reshape(OWOH, NF).sum(0)
    g_sumsq = co_sumsq[:FLAT].reshape(OWOH, NF).sum(0)
    n1 = B * OWOH
    m1 = g_sum / n1
    v1 = g_sumsq / n1 - m1 * m1
    alpha_f = bn1_g / jnp.sqrt(v1 + 1e-5)
    beta_f = bn1_b - m1 * alpha_f
    alpha = jnp.pad(jnp.tile(alpha_f * a0, OWOH), (0, FLATP - FLAT))
    beta = jnp.pad(jnp.tile(alpha_f, OWOH) * c0 * wsum[:FLAT]
                   + jnp.tile(beta_f, OWOH), (0, FLATP - FLAT))
    ab = jnp.zeros((8, FLATP), f32).at[0].set(alpha).at[1].set(beta)

    fc_w2 = fc_w.reshape(NF, OWOH, D).transpose(1, 0, 2).reshape(FLAT, D)
    fc_w2 = jnp.pad(fc_w2.astype(f32), ((0, FLATP - FLAT), (0, 0)))

    h, hst = _tc_fc(raw, ab, fc_w2, fc_b.reshape(1, D).astype(f32))

    m2 = hst[0] / B
    v2 = hst[1] / B - m2 * m2
    g2s = bn2_g / jnp.sqrt(v2 + 1e-5)
    bn2pack = jnp.zeros((8, D), f32).at[0].set(m2).at[1].set(g2s).at[2].set(bn2_b)

    bias = jnp.pad(ent_bias.reshape(1, NUM_ENT).astype(f32),
                   ((0, 0), (0, NPAD - NUM_ENT)))
    score = _tc_score(h, bn2pack, all_ent, bias)
    return score[:, :NUM_ENT]
